# baseline (speedup 1.0000x reference)
import functools

import jax
import jax.numpy as jnp
from jax import lax
from jax.experimental import pallas as pl
from jax.experimental.pallas import tpu as pltpu
from jax.experimental.pallas import tpu_sc as plsc

_B, _C, _H, _W = 4, 19, 512, 512
_HW = _H * _W
_LANES = 128
_ROWS = _HW // _LANES
_N = _B * _HW
_K = _N // 2
_NB = 4096
_SCALE = _NB / 16.0
_NC, _NS, _L = 2, 16, 16
_NW = _NC * _NS
_CHB = _HW // _NW
_VECB = _CHB // _L
_CHUNK = _N // _NW
_VECS = _CHUNK // _L
_BINV = _NB // _L
_BPW = _NB // _NS

_sc_mesh = plsc.VectorSubcoreMesh(core_axis_name="c", subcore_axis_name="s",
                                  num_cores=_NC, num_subcores=_NS)
_sc_params = pltpu.CompilerParams(needs_layout_passes=False)


def _ce_kernel(x_ref, t_ref, ce_ref):
    x = x_ref[0]
    t = t_ref[0]
    m = jnp.max(x, axis=0)
    s = jnp.sum(jnp.exp(x - m[None]), axis=0)
    lse = jnp.log(s) + m
    cls = jax.lax.broadcasted_iota(jnp.int32, (_C, _ROWS, _LANES), 0)
    xt = jnp.sum(jnp.where(cls == t[None], x, 0.0), axis=0)
    ce_ref[0] = lse - xt


def _ce_call(x4, t3, b):
    return pl.pallas_call(
        _ce_kernel,
        grid=(1,),
        in_specs=[
            pl.BlockSpec((1, _C, _ROWS, _LANES), lambda i, b=b: (b, 0, 0, 0)),
            pl.BlockSpec((1, _ROWS, _LANES), lambda i, b=b: (b, 0, 0)),
        ],
        out_specs=pl.BlockSpec((1, _ROWS, _LANES), lambda i: (0, 0, 0)),
        out_shape=jax.ShapeDtypeStruct((1, _ROWS, _LANES), jnp.float32),
        compiler_params=pltpu.CompilerParams(
            dimension_semantics=("arbitrary",),
        ),
    )(x4, t3)


@functools.partial(
    pl.kernel, mesh=_sc_mesh,
    out_type=jax.ShapeDtypeStruct((_NC, _NB), jnp.float32),
    scratch_types=[
        pltpu.VMEM((_CHB,), jnp.float32),
        pltpu.VMEM((_NB,), jnp.float32),
        pltpu.VMEM((_BPW,), jnp.float32),
        pltpu.VMEM((_NS, _BPW), jnp.float32),
        pltpu.VMEM_SHARED((_NS, _NB), jnp.float32),
    ],
    compiler_params=_sc_params,
)
def _sc_hist(ce_hbm, hist_hbm, data_v, hist_v, merge_v, mbuf_v, slots):
    c = lax.axis_index("c")
    s = lax.axis_index("s")
    wid = s * _NC + c
    zeros = jnp.zeros((_L,), jnp.float32)
    ones = jnp.ones((_L,), jnp.float32)

    pltpu.sync_copy(ce_hbm.at[pl.ds(wid * _CHB, _CHB)], data_v)

    def zb(i, _):
        hist_v[pl.ds(i * _L, _L)] = zeros
        return 0
    lax.fori_loop(0, _BINV, zb, 0)

    _UN = 8

    def hb(i, _):
        for u in range(_UN):
            v = data_v[pl.ds((i * _UN + u) * _L, _L)]
            idx = jnp.clip(v * _SCALE, 0.0, float(_NB - 1)).astype(jnp.int32)
            plsc.addupdate_scatter(hist_v, [idx], ones)
        return 0
    lax.fori_loop(0, _VECB // _UN, hb, 0)

    pltpu.sync_copy(hist_v, slots.at[s])
    plsc.subcore_barrier()

    pltpu.sync_copy(slots.at[:, pl.ds(s * _BPW, _BPW)], mbuf_v)

    def mr(i, _):
        sl = pl.ds(i * _L, _L)
        acc = mbuf_v[0, sl]
        for w in range(1, _NS):
            acc = acc + mbuf_v[w, sl]
        merge_v[sl] = acc
        return 0
    lax.fori_loop(0, _BPW // _L, mr, 0)

    pltpu.sync_copy(merge_v, hist_hbm.at[c, pl.ds(s * _BPW, _BPW)])


@functools.partial(
    pl.kernel, mesh=_sc_mesh,
    out_type=jax.ShapeDtypeStruct((_NC, _L), jnp.float32),
    scratch_types=[
        pltpu.VMEM((_CHUNK,), jnp.float32),
        pltpu.VMEM((_B, _NC, _NB), jnp.float32),
        pltpu.VMEM((_L,), jnp.float32),
        pltpu.VMEM_SHARED((_NS, _L), jnp.float32),
    ],
    compiler_params=_sc_params,
)
def _sc_select(ce0, ce1, ce2, ce3, h0, h1, h2, h3, out_hbm,
               data_v, hist_v, stage_v, slots):
    c = lax.axis_index("c")
    s = lax.axis_index("s")
    wid = s * _NC + c
    lane = lax.iota(jnp.int32, _L)

    for q, ce_hbm in enumerate((ce0, ce1, ce2, ce3)):
        pltpu.sync_copy(ce_hbm.at[pl.ds(wid * _CHB, _CHB)],
                        data_v.at[pl.ds(q * _CHB, _CHB)])
    for q, h_hbm in enumerate((h0, h1, h2, h3)):
        pltpu.sync_copy(h_hbm, hist_v.at[q])

    def sb(j, carry):
        tot_above, bstar_f = carry
        b = _BINV - 1 - j
        sl = pl.ds(b * _L, _L)
        v = hist_v[0, 0, sl]
        for q in range(_B):
            for cc in range(_NC):
                if q or cc:
                    v = v + hist_v[q, cc, sl]
        total = jnp.sum(v, axis=0)
        pre = plsc.cumsum(v)
        sfx = (total + tot_above) - pre + v
        nq = jnp.sum(jnp.where(sfx >= float(_K), 1.0, 0.0), axis=0)
        cand = jnp.where(nq > 0.0, (b * _L).astype(jnp.float32) + nq - 1.0, -1.0)
        return tot_above + total, jnp.maximum(bstar_f, cand)
    _, bstar_f = lax.fori_loop(0, _BINV, sb, (jnp.float32(0.0),
                                              jnp.float32(-1.0)))
    th = bstar_f * (1.0 / _SCALE)

    _UN = 8

    def rb(i, carry):
        sa, ca = carry
        for u in range(_UN):
            v = data_v[pl.ds((i * _UN + u) * _L, _L)]
            m = v >= th
            sa = sa + jnp.where(m, v, 0.0)
            ca = ca + jnp.where(m, 1.0, 0.0)
        return sa, ca
    sa, ca = lax.fori_loop(0, _VECS // _UN, rb,
                           (jnp.zeros((_L,), jnp.float32),
                            jnp.zeros((_L,), jnp.float32)))
    ssum = jnp.sum(sa, axis=0)
    scnt = jnp.sum(ca, axis=0)
    stage_v[...] = jnp.where(lane == 0, ssum, jnp.where(lane == 1, scnt, 0.0))
    pltpu.sync_copy(stage_v, slots.at[s])
    plsc.subcore_barrier()

    @pl.when(s == 0)
    def _():
        def ar(w, acc):
            pltpu.sync_copy(slots.at[w], stage_v)
            return acc + stage_v[...]
        acc = lax.fori_loop(0, _NS, ar, jnp.zeros((_L,), jnp.float32))
        stage_v[...] = acc
        pltpu.sync_copy(stage_v, out_hbm.at[c])


@functools.partial(jax.jit, static_argnames=())
def kernel(inputs, targets):
    x4 = inputs.reshape(_B, _C, _ROWS, _LANES)
    t3 = targets.astype(jnp.int32).reshape(_B, _ROWS, _LANES)
    ces = [_ce_call(x4, t3, b).reshape(_HW) for b in range(_B)]
    hists = [_sc_hist(ce) for ce in ces]
    parts = _sc_select(*ces, *hists)
    ssum = parts[0, 0] + parts[1, 0]
    scnt = parts[0, 1] + parts[1, 1]
    return ssum / scnt


# speedup vs baseline: 1.0573x; 1.0573x over previous
import functools

import jax
import jax.numpy as jnp
from jax import lax
from jax.experimental import pallas as pl
from jax.experimental.pallas import tpu as pltpu
from jax.experimental.pallas import tpu_sc as plsc

_B, _C, _H, _W = 4, 19, 512, 512
_HW = _H * _W
_LANES = 128
_ROWS = _HW // _LANES
_RA = 2048
_N = _B * _HW
_K = _N // 2
_NB = 4096
_SCALE = _NB / 16.0
_NC, _NS, _L = 2, 16, 16
_NW = _NC * _NS
_CHUNK = _N // _NW
_VECS = _CHUNK // _L
_BINV = _NB // _L
_BPW = _NB // _NS

_sc_mesh = plsc.VectorSubcoreMesh(core_axis_name="c", subcore_axis_name="s",
                                  num_cores=_NC, num_subcores=_NS)
_sc_params = pltpu.CompilerParams(needs_layout_passes=False)


def _ce_kernel(x_ref, t_ref, ce_ref):
    x = x_ref[0]
    t = t_ref[0]
    m = jnp.max(x, axis=0)
    s = jnp.sum(jnp.exp(x - m[None]), axis=0)
    lse = jnp.log(s) + m
    cls = jax.lax.broadcasted_iota(jnp.int32, (_C, _RA, _LANES), 0)
    xt = jnp.sum(jnp.where(cls == t[None], x, 0.0), axis=0)
    ce_ref[0] = lse - xt


@functools.partial(
    pl.kernel, mesh=_sc_mesh,
    out_type=jax.ShapeDtypeStruct((_NC, _NB), jnp.float32),
    scratch_types=[
        pltpu.VMEM((_CHUNK,), jnp.float32),
        pltpu.VMEM((_NB,), jnp.float32),
        pltpu.VMEM((_BPW,), jnp.float32),
        pltpu.VMEM((_NS, _BPW), jnp.float32),
        pltpu.VMEM_SHARED((_NS, _NB), jnp.float32),
    ],
    compiler_params=_sc_params,
)
def _sc_hist(ce_hbm, hist_hbm, data_v, hist_v, merge_v, mbuf_v, slots):
    c = lax.axis_index("c")
    s = lax.axis_index("s")
    wid = s * _NC + c
    zeros = jnp.zeros((_L,), jnp.float32)
    ones = jnp.ones((_L,), jnp.float32)

    pltpu.sync_copy(ce_hbm.at[pl.ds(wid * _CHUNK, _CHUNK)], data_v)

    def zb(i, _):
        hist_v[pl.ds(i * _L, _L)] = zeros
        return 0
    lax.fori_loop(0, _BINV, zb, 0)

    _UN = 8

    def hb(i, _):
        for u in range(_UN):
            v = data_v[pl.ds((i * _UN + u) * _L, _L)]
            idx = jnp.clip(v * _SCALE, 0.0, float(_NB - 1)).astype(jnp.int32)
            plsc.addupdate_scatter(hist_v, [idx], ones)
        return 0
    lax.fori_loop(0, _VECS // _UN, hb, 0)

    pltpu.sync_copy(hist_v, slots.at[s])
    plsc.subcore_barrier()

    pltpu.sync_copy(slots.at[:, pl.ds(s * _BPW, _BPW)], mbuf_v)

    def mr(i, _):
        sl = pl.ds(i * _L, _L)
        acc = mbuf_v[0, sl]
        for w in range(1, _NS):
            acc = acc + mbuf_v[w, sl]
        merge_v[sl] = acc
        return 0
    lax.fori_loop(0, _BPW // _L, mr, 0)

    pltpu.sync_copy(merge_v, hist_hbm.at[c, pl.ds(s * _BPW, _BPW)])


@functools.partial(
    pl.kernel, mesh=_sc_mesh,
    out_type=jax.ShapeDtypeStruct((_NC, _L), jnp.float32),
    scratch_types=[
        pltpu.VMEM((_CHUNK,), jnp.float32),
        pltpu.VMEM((_NC, _NB), jnp.float32),
        pltpu.VMEM((_L,), jnp.float32),
        pltpu.VMEM_SHARED((_NS, _L), jnp.float32),
    ],
    compiler_params=_sc_params,
)
def _sc_select(ce_hbm, hist_hbm, out_hbm, data_v, hist_v, stage_v, slots):
    c = lax.axis_index("c")
    s = lax.axis_index("s")
    wid = s * _NC + c
    lane = lax.iota(jnp.int32, _L)

    pltpu.sync_copy(ce_hbm.at[pl.ds(wid * _CHUNK, _CHUNK)], data_v)
    pltpu.sync_copy(hist_hbm, hist_v)

    def sb(j, carry):
        tot_above, bstar_f = carry
        b = _BINV - 1 - j
        v = hist_v[0, pl.ds(b * _L, _L)] + hist_v[1, pl.ds(b * _L, _L)]
        total = jnp.sum(v, axis=0)
        pre = plsc.cumsum(v)
        sfx = (total + tot_above) - pre + v
        nq = jnp.sum(jnp.where(sfx >= float(_K), 1.0, 0.0), axis=0)
        cand = jnp.where(nq > 0.0, (b * _L).astype(jnp.float32) + nq - 1.0, -1.0)
        return tot_above + total, jnp.maximum(bstar_f, cand)
    _, bstar_f = lax.fori_loop(0, _BINV, sb, (jnp.float32(0.0),
                                              jnp.float32(-1.0)))
    th = bstar_f * (1.0 / _SCALE)

    _UN = 8

    def rb(i, carry):
        sa, ca = carry
        for u in range(_UN):
            v = data_v[pl.ds((i * _UN + u) * _L, _L)]
            m = v >= th
            sa = sa + jnp.where(m, v, 0.0)
            ca = ca + jnp.where(m, 1.0, 0.0)
        return sa, ca
    sa, ca = lax.fori_loop(0, _VECS // _UN, rb,
                           (jnp.zeros((_L,), jnp.float32),
                            jnp.zeros((_L,), jnp.float32)))
    ssum = jnp.sum(sa, axis=0)
    scnt = jnp.sum(ca, axis=0)
    stage_v[...] = jnp.where(lane == 0, ssum, jnp.where(lane == 1, scnt, 0.0))
    pltpu.sync_copy(stage_v, slots.at[s])
    plsc.subcore_barrier()

    @pl.when(s == 0)
    def _():
        def ar(w, acc):
            pltpu.sync_copy(slots.at[w], stage_v)
            return acc + stage_v[...]
        acc = lax.fori_loop(0, _NS, ar, jnp.zeros((_L,), jnp.float32))
        stage_v[...] = acc
        pltpu.sync_copy(stage_v, out_hbm.at[c])


@functools.partial(jax.jit, static_argnames=())
def kernel(inputs, targets):
    x4 = inputs.reshape(_B, _C, _ROWS, _LANES)
    t3 = targets.astype(jnp.int32).reshape(_B, _ROWS, _LANES)
    nj = _ROWS // _RA
    ce = pl.pallas_call(
        _ce_kernel,
        grid=(_B, nj),
        in_specs=[
            pl.BlockSpec((1, _C, _RA, _LANES), lambda b, j: (b, 0, j, 0)),
            pl.BlockSpec((1, _RA, _LANES), lambda b, j: (b, j, 0)),
        ],
        out_specs=pl.BlockSpec((1, _RA, _LANES), lambda b, j: (b, j, 0)),
        out_shape=jax.ShapeDtypeStruct((_B, _ROWS, _LANES), jnp.float32),
        compiler_params=pltpu.CompilerParams(
            dimension_semantics=("arbitrary", "arbitrary"),
        ),
    )(x4, t3)
    cef = ce.reshape(_N)
    hist = _sc_hist(cef)
    parts = _sc_select(cef, hist)
    ssum = parts[0, 0] + parts[1, 0]
    scnt = parts[0, 1] + parts[1, 1]
    return ssum / scnt


# speedup vs baseline: 1.0666x; 1.0088x over previous
import functools

import jax
import jax.numpy as jnp
from jax import lax
from jax.experimental import pallas as pl
from jax.experimental.pallas import tpu as pltpu
from jax.experimental.pallas import tpu_sc as plsc

_B, _C, _H, _W = 4, 19, 512, 512
_HW = _H * _W
_LANES = 128
_ROWS = _HW // _LANES
_RA = 2048
_N = _B * _HW
_K = _N // 2
_NB = 4096
_SCALE = _NB / 16.0
_NC, _NS, _L = 2, 16, 16
_NW = _NC * _NS
_CHUNK = _N // _NW
_VECS = _CHUNK // _L
_BINV = _NB // _L
_BPW = _NB // _NS

_sc_mesh = plsc.VectorSubcoreMesh(core_axis_name="c", subcore_axis_name="s",
                                  num_cores=_NC, num_subcores=_NS)
_sc_params = pltpu.CompilerParams(needs_layout_passes=False)


def _ce_kernel(x_ref, t_ref, ce_ref):
    x = x_ref[0]
    t = t_ref[0]
    m = jnp.max(x, axis=0)
    s = jnp.sum(jnp.exp(x - m[None]), axis=0)
    lse = jnp.log(s) + m
    cls = jax.lax.broadcasted_iota(jnp.int32, (_C, _RA, _LANES), 0)
    xt = jnp.sum(jnp.where(cls == t[None], x, 0.0), axis=0)
    ce_ref[0] = lse - xt


@functools.partial(
    pl.kernel, mesh=_sc_mesh,
    out_type=jax.ShapeDtypeStruct((_NC, _NB), jnp.float32),
    scratch_types=[
        pltpu.VMEM((_CHUNK,), jnp.float32),
        pltpu.VMEM((_NB,), jnp.float32),
        pltpu.VMEM((_BPW,), jnp.float32),
        pltpu.VMEM((_NS, _BPW), jnp.float32),
        pltpu.VMEM_SHARED((_NS, _NB), jnp.float32),
        pltpu.SemaphoreType.DMA,
    ],
    compiler_params=_sc_params,
)
def _sc_hist(ce_hbm, hist_hbm, data_v, hist_v, merge_v, mbuf_v, slots, sem):
    c = lax.axis_index("c")
    s = lax.axis_index("s")
    wid = s * _NC + c
    zeros = jnp.zeros((_L,), jnp.float32)
    ones = jnp.ones((_L,), jnp.float32)

    cp = pltpu.async_copy(ce_hbm.at[pl.ds(wid * _CHUNK, _CHUNK)], data_v, sem)

    def zb(i, _):
        hist_v[pl.ds(i * _L, _L)] = zeros
        return 0
    lax.fori_loop(0, _BINV, zb, 0)
    cp.wait()

    _UN = 8

    def hb(i, _):
        for u in range(_UN):
            v = data_v[pl.ds((i * _UN + u) * _L, _L)]
            idx = jnp.clip(v * _SCALE, 0.0, float(_NB - 1)).astype(jnp.int32)
            plsc.addupdate_scatter(hist_v, [idx], ones)
        return 0
    lax.fori_loop(0, _VECS // _UN, hb, 0)

    pltpu.sync_copy(hist_v, slots.at[s])
    plsc.subcore_barrier()

    pltpu.sync_copy(slots.at[:, pl.ds(s * _BPW, _BPW)], mbuf_v)

    def mr(i, _):
        sl = pl.ds(i * _L, _L)
        acc = mbuf_v[0, sl]
        for w in range(1, _NS):
            acc = acc + mbuf_v[w, sl]
        merge_v[sl] = acc
        return 0
    lax.fori_loop(0, _BPW // _L, mr, 0)

    pltpu.sync_copy(merge_v, hist_hbm.at[c, pl.ds(s * _BPW, _BPW)])


@functools.partial(
    pl.kernel, mesh=_sc_mesh,
    out_type=jax.ShapeDtypeStruct((_NC, _L), jnp.float32),
    scratch_types=[
        pltpu.VMEM((_CHUNK,), jnp.float32),
        pltpu.VMEM((_NC, _NB), jnp.float32),
        pltpu.VMEM((_L,), jnp.float32),
        pltpu.VMEM_SHARED((_NS, _L), jnp.float32),
        pltpu.SemaphoreType.DMA,
    ],
    compiler_params=_sc_params,
)
def _sc_select(ce_hbm, hist_hbm, out_hbm, data_v, hist_v, stage_v, slots, sem):
    c = lax.axis_index("c")
    s = lax.axis_index("s")
    wid = s * _NC + c
    lane = lax.iota(jnp.int32, _L)

    cp = pltpu.async_copy(ce_hbm.at[pl.ds(wid * _CHUNK, _CHUNK)], data_v, sem)
    pltpu.sync_copy(hist_hbm, hist_v)

    def sb(j, carry):
        tot_above, bstar_f = carry
        b = _BINV - 1 - j
        v = hist_v[0, pl.ds(b * _L, _L)] + hist_v[1, pl.ds(b * _L, _L)]
        total = jnp.sum(v, axis=0)
        pre = plsc.cumsum(v)
        sfx = (total + tot_above) - pre + v
        nq = jnp.sum(jnp.where(sfx >= float(_K), 1.0, 0.0), axis=0)
        cand = jnp.where(nq > 0.0, (b * _L).astype(jnp.float32) + nq - 1.0, -1.0)
        return tot_above + total, jnp.maximum(bstar_f, cand)
    _, bstar_f = lax.fori_loop(0, _BINV, sb, (jnp.float32(0.0),
                                              jnp.float32(-1.0)))
    th = bstar_f * (1.0 / _SCALE)
    cp.wait()

    _UN = 8

    def rb(i, carry):
        sa, ca = carry
        for u in range(_UN):
            v = data_v[pl.ds((i * _UN + u) * _L, _L)]
            m = v >= th
            sa = sa + jnp.where(m, v, 0.0)
            ca = ca + jnp.where(m, 1.0, 0.0)
        return sa, ca
    sa, ca = lax.fori_loop(0, _VECS // _UN, rb,
                           (jnp.zeros((_L,), jnp.float32),
                            jnp.zeros((_L,), jnp.float32)))
    ssum = jnp.sum(sa, axis=0)
    scnt = jnp.sum(ca, axis=0)
    stage_v[...] = jnp.where(lane == 0, ssum, jnp.where(lane == 1, scnt, 0.0))
    pltpu.sync_copy(stage_v, slots.at[s])
    plsc.subcore_barrier()

    @pl.when(s == 0)
    def _():
        def ar(w, acc):
            pltpu.sync_copy(slots.at[w], stage_v)
            return acc + stage_v[...]
        acc = lax.fori_loop(0, _NS, ar, jnp.zeros((_L,), jnp.float32))
        stage_v[...] = acc
        pltpu.sync_copy(stage_v, out_hbm.at[c])


@functools.partial(jax.jit, static_argnames=())
def kernel(inputs, targets):
    x4 = inputs.reshape(_B, _C, _ROWS, _LANES)
    t3 = targets.astype(jnp.int32).reshape(_B, _ROWS, _LANES)
    nj = _ROWS // _RA
    ce = pl.pallas_call(
        _ce_kernel,
        grid=(_B, nj),
        in_specs=[
            pl.BlockSpec((1, _C, _RA, _LANES), lambda b, j: (b, 0, j, 0)),
            pl.BlockSpec((1, _RA, _LANES), lambda b, j: (b, j, 0)),
        ],
        out_specs=pl.BlockSpec((1, _RA, _LANES), lambda b, j: (b, j, 0)),
        out_shape=jax.ShapeDtypeStruct((_B, _ROWS, _LANES), jnp.float32),
        compiler_params=pltpu.CompilerParams(
            dimension_semantics=("arbitrary", "arbitrary"),
        ),
    )(x4, t3)
    cef = ce.reshape(_N)
    hist = _sc_hist(cef)
    parts = _sc_select(cef, hist)
    ssum = parts[0, 0] + parts[1, 0]
    scnt = parts[0, 1] + parts[1, 1]
    return ssum / scnt
